# baseline (speedup 1.0000x reference)
import functools

import jax
import jax.numpy as jnp
import numpy as np
from jax import lax
from jax.experimental import pallas as pl
from jax.experimental.pallas import tpu as pltpu

_GAMMA = 0.99
_ATOMS = 51
_VMIN = -10.0
_VMAX = 10.0
_NACT = 18
_DELTA = (_VMAX - _VMIN) / (_ATOMS - 1)
_AA = _NACT * _ATOMS


def _consts():
    a_of = np.arange(_AA) // _ATOMS
    i_of = np.arange(_AA) % _ATOMS
    sup = (_VMIN + _DELTA * i_of).astype(np.float32)
    w1 = np.zeros((_AA, 2 * _NACT), np.float32)
    w1[np.arange(_AA), a_of] = 1.0
    w1[np.arange(_AA), _NACT + a_of] = sup
    e = np.zeros((_NACT, _AA), np.float32)
    e[a_of, np.arange(_AA)] = 1.0
    p = np.zeros((_AA, _ATOMS), np.float32)
    p[np.arange(_AA), i_of] = 1.0
    return (jnp.asarray(w1, jnp.bfloat16), jnp.asarray(e, jnp.bfloat16),
            jnp.asarray(p, jnp.bfloat16))


def _body(cur_ref, nxt_ref, rew_ref, act_ref, mask_ref, w1_ref, e_ref, p_ref,
          out_ref, *, nb, bk, total_b):
    i = pl.program_id(0)

    nxt = nxt_ref[...]
    rew = rew_ref[...]
    act = act_ref[...]
    msk = mask_ref[...]

    ex = jnp.exp(nxt)
    zq = jnp.dot(ex, w1_ref[...], preferred_element_type=jnp.float32)
    z = zq[:, :_NACT]
    q = zq[:, _NACT:] / z
    qmax = jnp.max(q, axis=-1, keepdims=True)
    w = (q >= qmax).astype(jnp.float32)
    wexp = jnp.dot(w.astype(jnp.bfloat16), e_ref[...],
                   preferred_element_type=jnp.float32).astype(jnp.bfloat16)
    ex_sel = jnp.dot(ex * wexp, p_ref[...], preferred_element_type=jnp.float32)
    z_sel = jnp.sum(w * z, axis=-1, keepdims=True)
    p = ex_sel / z_sel
    maskb = msk > 0
    p = jnp.where(maskb, p, jnp.float32(1.0 / _ATOMS))

    maskf = maskb.astype(jnp.float32)
    atom2 = lax.broadcasted_iota(jnp.int32, (bk, _ATOMS), 1).astype(jnp.float32)
    sup2 = _VMIN + _DELTA * atom2
    tz = jnp.clip(rew + _GAMMA * sup2 * maskf, _VMIN, _VMAX)
    bpos = (tz - _VMIN) / _DELTA
    l = jnp.floor(bpos)
    f = bpos - l
    lidx = l.astype(jnp.int32)
    uidx = jnp.minimum(lidx + 1, _ATOMS - 1)

    cur = cur_ref[...]
    aidx = lax.broadcasted_iota(jnp.int32, (bk, _NACT), 1)
    wact = (aidx == act).astype(jnp.bfloat16)
    cexp = jnp.dot(wact, e_ref[...],
                   preferred_element_type=jnp.float32).astype(jnp.bfloat16)
    c_sel = jnp.dot(cur * cexp, p_ref[...], preferred_element_type=jnp.float32)
    cmax = jnp.max(c_sel, axis=-1, keepdims=True)
    csh = c_sel - cmax
    logc = csh - jnp.log(jnp.sum(jnp.exp(csh), axis=-1, keepdims=True))

    logc_l = jnp.take_along_axis(logc, lidx, axis=-1)
    logc_u = jnp.take_along_axis(logc, uidx, axis=-1)
    t = (1.0 - f) * logc_l + f * logc_u
    partial = -jnp.sum(p * t)

    @pl.when(i == 0)
    def _init():
        out_ref[0, 0] = 0.0

    out_ref[0, 0] += partial

    @pl.when(i == nb - 1)
    def _fin():
        out_ref[0, 0] = out_ref[0, 0] / jnp.float32(total_b)


def kernel(current_logits, next_logits, rewards, actions, non_final_mask):
    b = current_logits.shape[0]
    bk = 128
    assert b % bk == 0
    nb = b // bk
    w1, e, p = _consts()

    out = pl.pallas_call(
        functools.partial(_body, nb=nb, bk=bk, total_b=b),
        grid=(nb,),
        in_specs=[
            pl.BlockSpec((bk, _AA), lambda i: (i, 0)),
            pl.BlockSpec((bk, _AA), lambda i: (i, 0)),
            pl.BlockSpec((bk, 1), lambda i: (i, 0)),
            pl.BlockSpec((bk, 1), lambda i: (i, 0)),
            pl.BlockSpec((bk, 1), lambda i: (i, 0)),
            pl.BlockSpec((_AA, 2 * _NACT), lambda i: (0, 0)),
            pl.BlockSpec((_NACT, _AA), lambda i: (0, 0)),
            pl.BlockSpec((_AA, _ATOMS), lambda i: (0, 0)),
        ],
        out_specs=pl.BlockSpec((1, 1), lambda i: (0, 0), memory_space=pltpu.SMEM),
        out_shape=jax.ShapeDtypeStruct((1, 1), jnp.float32),
    )(
        current_logits.reshape(b, _AA).astype(jnp.bfloat16),
        next_logits.reshape(b, _AA).astype(jnp.bfloat16),
        rewards.reshape(b, 1),
        actions.reshape(b, 1),
        non_final_mask.reshape(b, 1),
        w1,
        e,
        p,
    )
    return out[0, 0]


# speedup vs baseline: 1.1876x; 1.1876x over previous
import functools

import jax
import jax.numpy as jnp
import numpy as np
from jax import lax
from jax.experimental import pallas as pl
from jax.experimental.pallas import tpu as pltpu

_GAMMA = 0.99
_ATOMS = 51
_VMIN = -10.0
_VMAX = 10.0
_NACT = 18
_DELTA = (_VMAX - _VMIN) / (_ATOMS - 1)
_AA = _NACT * _ATOMS


def _consts():
    a_of = np.arange(_AA) // _ATOMS
    i_of = np.arange(_AA) % _ATOMS
    sup = (_VMIN + _DELTA * i_of).astype(np.float32)
    w1 = np.zeros((_AA, 2 * _NACT), np.float32)
    w1[np.arange(_AA), a_of] = 1.0
    w1[np.arange(_AA), _NACT + a_of] = sup
    e = np.zeros((_NACT, _AA), np.float32)
    e[a_of, np.arange(_AA)] = 1.0
    p = np.zeros((_AA, _ATOMS), np.float32)
    p[np.arange(_AA), i_of] = 1.0
    return (jnp.asarray(w1, jnp.bfloat16), jnp.asarray(e, jnp.bfloat16),
            jnp.asarray(p, jnp.bfloat16))


def _body(cur_ref, nxt_ref, rew_ref, act_ref, mask_ref, w1_ref, e_ref, p_ref,
          out_ref, *, nb, bk, total_b):
    i = pl.program_id(0)

    nxt = nxt_ref[...]
    rew = rew_ref[...]
    act = act_ref[...]
    msk = mask_ref[...]

    ex = jnp.exp(nxt)
    zq = jnp.dot(ex, w1_ref[...], preferred_element_type=jnp.float32)
    z = zq[:, :_NACT]
    q = zq[:, _NACT:] / z
    qmax = jnp.max(q, axis=-1, keepdims=True)
    w = (q >= qmax).astype(jnp.float32)
    wexp = jnp.dot(w.astype(jnp.bfloat16), e_ref[...],
                   preferred_element_type=jnp.float32).astype(jnp.bfloat16)
    ex_sel = jnp.dot(ex * wexp, p_ref[...], preferred_element_type=jnp.float32)
    z_sel = jnp.sum(w * z, axis=-1, keepdims=True)
    p = ex_sel / z_sel
    maskb = msk > 0
    p = jnp.where(maskb, p, jnp.float32(1.0 / _ATOMS))

    maskf = maskb.astype(jnp.float32)
    atom2 = lax.broadcasted_iota(jnp.int32, (bk, _ATOMS), 1).astype(jnp.float32)
    sup2 = _VMIN + _DELTA * atom2
    tz = jnp.clip(rew + _GAMMA * sup2 * maskf, _VMIN, _VMAX)
    bpos = (tz - _VMIN) / _DELTA
    l = jnp.floor(bpos)
    f = bpos - l
    lidx = l.astype(jnp.int32)
    uidx = jnp.minimum(lidx + 1, _ATOMS - 1)

    cur = cur_ref[...]
    aidx = lax.broadcasted_iota(jnp.int32, (bk, _NACT), 1)
    wact = (aidx == act).astype(jnp.bfloat16)
    cexp = jnp.dot(wact, e_ref[...],
                   preferred_element_type=jnp.float32).astype(jnp.bfloat16)
    c_sel = jnp.dot(cur * cexp, p_ref[...], preferred_element_type=jnp.float32)
    cmax = jnp.max(c_sel, axis=-1, keepdims=True)
    csh = c_sel - cmax
    logc = csh - jnp.log(jnp.sum(jnp.exp(csh), axis=-1, keepdims=True))

    logc_l = jnp.take_along_axis(logc, lidx, axis=-1)
    logc_u = jnp.take_along_axis(logc, uidx, axis=-1)
    t = (1.0 - f) * logc_l + f * logc_u
    partial = -jnp.sum(p * t)

    @pl.when(i == 0)
    def _init():
        out_ref[0, 0] = 0.0

    out_ref[0, 0] += partial

    @pl.when(i == nb - 1)
    def _fin():
        out_ref[0, 0] = out_ref[0, 0] / jnp.float32(total_b)


def kernel(current_logits, next_logits, rewards, actions, non_final_mask):
    b = current_logits.shape[0]
    bk = 256
    assert b % bk == 0
    nb = b // bk
    w1, e, p = _consts()

    out = pl.pallas_call(
        functools.partial(_body, nb=nb, bk=bk, total_b=b),
        grid=(nb,),
        in_specs=[
            pl.BlockSpec((bk, _AA), lambda i: (i, 0)),
            pl.BlockSpec((bk, _AA), lambda i: (i, 0)),
            pl.BlockSpec((bk, 1), lambda i: (i, 0)),
            pl.BlockSpec((bk, 1), lambda i: (i, 0)),
            pl.BlockSpec((bk, 1), lambda i: (i, 0)),
            pl.BlockSpec((_AA, 2 * _NACT), lambda i: (0, 0)),
            pl.BlockSpec((_NACT, _AA), lambda i: (0, 0)),
            pl.BlockSpec((_AA, _ATOMS), lambda i: (0, 0)),
        ],
        out_specs=pl.BlockSpec((1, 1), lambda i: (0, 0), memory_space=pltpu.SMEM),
        out_shape=jax.ShapeDtypeStruct((1, 1), jnp.float32),
    )(
        current_logits.reshape(b, _AA).astype(jnp.bfloat16),
        next_logits.reshape(b, _AA).astype(jnp.bfloat16),
        rewards.reshape(b, 1),
        actions.reshape(b, 1),
        non_final_mask.reshape(b, 1),
        w1,
        e,
        p,
    )
    return out[0, 0]
